# baseline (speedup 1.0000x reference)
import functools

import jax
import jax.numpy as jnp
from jax import lax
from jax.experimental import pallas as pl
from jax.experimental.pallas import tpu as pltpu
from jax.experimental.pallas import tpu_sc as plsc

MARGIN = 0.2
EPS = 1e-12

V, D = 16384, 128
DP = D // 2
B = 32768
NC, NS, L = 2, 16, 16
NW = NC * NS
TW = B // NW
CH = 64
NCHUNK = TW // CH
NRING = 3
IDX_ROWS = B // CH
RPT = V // NS // NC
SB = 64
NB = RPT // SB
MAGIC = 0x5CAB7E57


def _sqrt16(x):
    i = plsc.bitcast(x, jnp.int32)
    y = plsc.bitcast(jnp.int32(0x5F3759DF) - (i >> 1), jnp.float32)
    xh = x * 0.5
    y = y * (1.5 - xh * y * y)
    y = y * (1.5 - xh * y * y)
    y = y * (1.5 - xh * y * y)
    return x * y


def _unpack2(x_i32):
    ab = plsc.bitcast(x_i32, jnp.bfloat16)
    return plsc.unpack(ab, format=plsc.PackFormat.INTERLEAVED)


def _tl_body(emb, aidx, pidx, nidx,
             out_ap, out_an, out_td, out_part, tab, flag,
             aidx_v, pidx_v, nidx_v,
             stage0, stage1, stage2, stage3, pack0, pack1, flag_v,
             bufs_flat, ap_v, an_v, loss_v, sems, stage_sem, wr_sem,
             idx_sem):
    cid = lax.axis_index("c")
    sid = lax.axis_index("s")
    wid = sid * NC + cid
    base = wid * TW

    hi1 = pltpu.async_copy(aidx.at[pl.ds(wid * NCHUNK, NCHUNK)], aidx_v,
                           idx_sem)
    hi2 = pltpu.async_copy(pidx.at[pl.ds(wid * NCHUNK, NCHUNK)], pidx_v,
                           idx_sem)
    hi3 = pltpu.async_copy(nidx.at[pl.ds(wid * NCHUNK, NCHUNK)], nidx_v,
                           idx_sem)

    row0 = (cid * NS + sid) * RPT
    stages = (stage0, stage1, stage2, stage3)
    packs = (pack0, pack1)

    def fire_stage(b):
        return pltpu.async_copy(
            emb.at[pl.ds(row0 + b * SB, SB)], stages[b % 4], stage_sem)

    hss = [None] * NB
    for b in range(min(3, NB)):
        hss[b] = fire_stage(b)
    hw = [None, None]
    for b in range(NB):
        hss[b].wait()
        if b + 3 < NB:
            hss[b + 3] = fire_stage(b + 3)
        stage = stages[b % 4]
        pack_buf = packs[b % 2]
        if hw[b % 2] is not None:
            hw[b % 2].wait()

        def conv_rows(r, _, stage=stage, pack_buf=pack_buf):
            for rr in range(2):
                for j in range(4):
                    lo = stage[r * 2 + rr, pl.ds(j * 32, L)]
                    hi = stage[r * 2 + rr, pl.ds(j * 32 + L, L)]
                    packed = plsc.pack(
                        lo, hi, format=plsc.PackFormat.INTERLEAVED)
                    pack_buf[r * 2 + rr, pl.ds(j * L, L)] = plsc.bitcast(
                        packed, jnp.int32)
            return 0

        lax.fori_loop(0, SB // 2, conv_rows, 0, unroll=4)
        hw[b % 2] = pltpu.async_copy(
            pack_buf, tab.at[pl.ds(row0 + b * SB, SB)], wr_sem)
    for h in hw:
        if h is not None:
            h.wait()

    plsc.subcore_barrier()
    flag_v[...] = jnp.full((L,), MAGIC, dtype=jnp.int32)
    pltpu.sync_copy(flag_v, flag.at[cid])

    def poll_cond(x):
        return jnp.any(x != MAGIC)

    def poll_body(x):
        pltpu.sync_copy(flag.at[1 - cid], flag_v)
        return flag_v[...]

    lax.while_loop(poll_cond, poll_body, jnp.zeros((L,), jnp.int32))
    hi1.wait()
    hi2.wait()
    hi3.wait()

    iota = lax.iota(jnp.int32, L)
    bufs = tuple(tuple(bufs_flat[s * 3:s * 3 + 3]) + (sems[s],)
                 for s in range(NRING))

    def fire(c):
        a_buf, p_buf, n_buf, sem = bufs[c % NRING]
        return (pltpu.async_copy(tab.at[aidx_v.at[c]], a_buf, sem),
                pltpu.async_copy(tab.at[pidx_v.at[c]], p_buf, sem),
                pltpu.async_copy(tab.at[nidx_v.at[c]], n_buf, sem))

    def chunk_compute(c, loss_acc):
        a_buf, p_buf, n_buf, _ = bufs[c % NRING]

        def group_body(g, acc):
            row = jnp.full((L,), g * L, dtype=jnp.int32) + iota

            def d_body(dd, carry):
                ap0, ap1, an0, an1, col = carry
                for _ in range(8):
                    xa = plsc.load_gather(a_buf, [row, col])
                    xp = plsc.load_gather(p_buf, [row, col])
                    xn = plsc.load_gather(n_buf, [row, col])
                    av0, av1 = _unpack2(xa)
                    pv0, pv1 = _unpack2(xp)
                    nv0, nv1 = _unpack2(xn)
                    dap0 = av0 - pv0 + EPS
                    dap1 = av1 - pv1 + EPS
                    dan0 = av0 - nv0 + EPS
                    dan1 = av1 - nv1 + EPS
                    ap0 = ap0 + dap0 * dap0
                    ap1 = ap1 + dap1 * dap1
                    an0 = an0 + dan0 * dan0
                    an1 = an1 + dan1 * dan1
                    col = (col + 1) & (DP - 1)
                return ap0, ap1, an0, an1, col

            z = jnp.zeros((L,), jnp.float32)
            ap0, ap1, an0, an1, _ = lax.fori_loop(
                0, DP // 8, d_body, (z, z, z, z, iota))
            ap = _sqrt16(ap0 + ap1)
            an = _sqrt16(an0 + an1)
            off = c * CH + g * L
            ap_v[pl.ds(off, L)] = ap
            an_v[pl.ds(off, L)] = an
            return acc + jnp.maximum(ap - an + MARGIN, 0.0)

        return lax.fori_loop(0, CH // L, group_body, loss_acc)

    loss_acc = jnp.zeros((L,), jnp.float32)
    handles = {}
    for c in range(NRING - 1):
        handles[c] = fire(c)
    for c in range(NCHUNK):
        for h in handles.pop(c):
            h.wait()
        nxt = c + NRING - 1
        if nxt < NCHUNK:
            handles[nxt] = fire(nxt)
        loss_acc = chunk_compute(c, loss_acc)

    loss_v[...] = loss_acc
    pltpu.sync_copy(loss_v, out_part.at[wid])
    pltpu.sync_copy(ap_v, out_ap.at[pl.ds(base, TW)])
    pltpu.sync_copy(an_v, out_an.at[pl.ds(base, TW)])
    pltpu.sync_copy(ap_v, out_td.at[pl.ds(base, TW)])
    pltpu.sync_copy(an_v, out_td.at[pl.ds(B + base, TW)])


_tl_kernel = functools.partial(
    pl.kernel,
    mesh=plsc.VectorSubcoreMesh(core_axis_name="c", subcore_axis_name="s"),
    compiler_params=pltpu.CompilerParams(
        needs_layout_passes=False, use_tc_tiling_on_sc=False),
    out_type=[
        jax.ShapeDtypeStruct((B,), jnp.float32),
        jax.ShapeDtypeStruct((B,), jnp.float32),
        jax.ShapeDtypeStruct((2 * B,), jnp.float32),
        jax.ShapeDtypeStruct((NW, L), jnp.float32),
        jax.ShapeDtypeStruct((V, DP), jnp.int32),
        jax.ShapeDtypeStruct((NC, L), jnp.int32),
    ],
    scratch_types=[
        pltpu.VMEM((NCHUNK, CH), jnp.int32),
        pltpu.VMEM((NCHUNK, CH), jnp.int32),
        pltpu.VMEM((NCHUNK, CH), jnp.int32),
        pltpu.VMEM((SB, D), jnp.float32),
        pltpu.VMEM((SB, D), jnp.float32),
        pltpu.VMEM((SB, D), jnp.float32),
        pltpu.VMEM((SB, D), jnp.float32),
        pltpu.VMEM((SB, DP), jnp.int32),
        pltpu.VMEM((SB, DP), jnp.int32),
        pltpu.VMEM((L,), jnp.int32),
        [pltpu.VMEM((CH, DP), jnp.int32) for _ in range(3 * NRING)],
        pltpu.VMEM((TW,), jnp.float32),
        pltpu.VMEM((TW,), jnp.float32),
        pltpu.VMEM((L,), jnp.float32),
        [pltpu.SemaphoreType.DMA for _ in range(NRING)],
        pltpu.SemaphoreType.DMA,
        pltpu.SemaphoreType.DMA,
        pltpu.SemaphoreType.DMA,
    ],
)(_tl_body)


def kernel(embeddings, target, triplets):
    del target
    aidx = triplets[:, 0].reshape(IDX_ROWS, CH)
    pidx = triplets[:, 1].reshape(IDX_ROWS, CH)
    nidx = triplets[:, 2].reshape(IDX_ROWS, CH)
    out_ap, out_an, out_td, out_part, _, _ = _tl_kernel(
        embeddings, aidx, pidx, nidx)
    loss = jnp.sum(out_part) / B
    tt = jnp.concatenate(
        [jnp.ones((B,), jnp.float32), jnp.zeros((B,), jnp.float32)])
    return loss, out_ap, out_an, out_td, tt
